# baseline (speedup 1.0000x reference)
import functools

import jax
import jax.numpy as jnp
from jax.experimental import pallas as pl
from jax.experimental.pallas import tpu as pltpu

K = 4
NHID = 16
NFEAT = 128
S0 = 25
S1 = 10
B = 2048
LDIM = 32
FEAT1 = K * NHID


def _leaky(x):
    return jnp.where(x >= 0, x, 0.2 * x)


def _gat_kernel(nodes_ref, neis_ref, sneis_ref, w1m_ref, a1lrep_ref, r_ref,
                sum4_ref, bcast_ref, bc4_ref, slabsum_ref,
                w2m_ref, a2l_ref, a2r_ref, wl_ref, bl_ref, out_ref,
                slab_ref, sems, *, bb):
    n1 = bb * S0
    f32 = jnp.float32

    w1m = w1m_ref[...]

    copies = [
        pltpu.make_async_copy(
            sneis_ref.at[:, s, :],
            slab_ref.at[pl.ds(s * n1, n1), :],
            sems.at[s],
        )
        for s in range(S1)
    ]
    for c in copies:
        c.start()
    for c in copies:
        c.wait()

    hn_s = [
        jnp.dot(slab_ref[pl.ds(s * n1, n1), :], w1m, preferred_element_type=f32)
        for s in range(S1)
    ]
    hnw = jnp.concatenate(hn_s, axis=1)

    es = jnp.dot(neis_ref[...], a1lrep_ref[...], preferred_element_type=f32)
    en = jnp.dot(hnw, r_ref[...], preferred_element_type=f32)
    e = _leaky(es + en)
    p = jnp.exp(jnp.minimum(e, 60.0))

    den = jnp.dot(p, sum4_ref[...], preferred_element_type=f32)
    rden = 1.0 / den
    rdenb = jnp.dot(rden, bc4_ref[...], preferred_element_type=f32)

    pb = jnp.dot(p, bcast_ref[...], preferred_element_type=f32)
    weighted = pb * hnw
    acc = jnp.dot(weighted, slabsum_ref[...], preferred_element_type=f32)

    out1 = acc * rdenb
    g1 = jnp.where(out1 > 0, out1, jnp.exp(jnp.minimum(out1, 0.0)) - 1.0)

    w2m = w2m_ref[...]
    hs2 = jnp.dot(nodes_ref[...], w2m, preferred_element_type=f32)
    hn2 = jnp.dot(g1, w2m, preferred_element_type=f32)

    es2 = jnp.dot(hs2, a2l_ref[...], preferred_element_type=f32)
    en2 = jnp.dot(hn2, a2r_ref[...], preferred_element_type=f32)

    nid = jax.lax.broadcasted_iota(jnp.int32, (bb, n1), 1) // S0
    bid = jax.lax.broadcasted_iota(jnp.int32, (bb, n1), 0)
    seg = jnp.where(nid == bid, 1.0, 0.0)
    rmod = jax.lax.broadcasted_iota(jnp.int32, (n1, S0), 0) % S0
    jidx = jax.lax.broadcasted_iota(jnp.int32, (n1, S0), 1)
    t = jnp.where(rmod == jidx, 1.0, 0.0)

    e2m = jnp.dot(seg, en2 * t, preferred_element_type=f32)
    e2 = _leaky(es2 + e2m)
    p2 = jnp.exp(jnp.minimum(e2, 60.0))
    alpha2 = p2 / jnp.sum(p2, axis=1, keepdims=True)

    alpha_tiled = jnp.concatenate([alpha2] * bb, axis=1)
    seg_w = alpha_tiled * seg
    out2 = jnp.dot(seg_w, hn2, preferred_element_type=f32)

    z = jnp.dot(out2, wl_ref[...], preferred_element_type=f32) + bl_ref[...]
    out_ref[...] = 1.0 / (1.0 + jnp.exp(-z))


def kernel(nodes, node_neis, node_second_neis, W1, a1, W2, a2, Wl, bl):
    bb = 32
    grid = (B // bb,)
    f32 = jnp.float32

    w1m = jnp.transpose(W1, (1, 0, 2)).reshape(NFEAT, FEAT1)
    a1l = a1[:, :NHID]
    a1r = a1[:, NHID:]
    eye_k = jnp.eye(K, dtype=f32)
    eye_s = jnp.eye(S1, dtype=f32)

    u = jnp.einsum('kfo,ko->fk', W1, a1l)
    erep = jnp.repeat(eye_k, S1, axis=0).T
    a1lrep = u @ erep

    r6 = (a1r[None, :, :, None, None]
          * eye_k[None, :, None, :, None]
          * eye_s[:, None, None, None, :])
    r_m = r6.reshape(S1 * FEAT1, K * S1)

    sum4 = jnp.repeat(eye_k, S1, axis=0)
    y = (eye_k[:, None, None, :, None]
         * eye_s[None, :, :, None, None]
         * jnp.ones((1, 1, 1, 1, NHID), f32))
    bcast = y.reshape(K * S1, S1 * FEAT1)
    bc4 = jnp.repeat(eye_k, NHID, axis=1)
    slabsum = jnp.tile(jnp.eye(FEAT1, dtype=f32), (S1, 1))

    w2m = W2[0]
    a2l = a2[0, :LDIM].reshape(LDIM, 1)
    a2r = a2[0, LDIM:].reshape(LDIM, 1)
    bl2 = bl.reshape(1, 1)

    out = pl.pallas_call(
        functools.partial(_gat_kernel, bb=bb),
        grid=grid,
        in_specs=[
            pl.BlockSpec((bb, FEAT1), lambda i: (i, 0)),
            pl.BlockSpec((bb * S0, NFEAT), lambda i: (i, 0)),
            pl.BlockSpec((bb * S0, S1, NFEAT), lambda i: (i, 0, 0)),
            pl.BlockSpec((NFEAT, FEAT1), lambda i: (0, 0)),
            pl.BlockSpec((NFEAT, K * S1), lambda i: (0, 0)),
            pl.BlockSpec((S1 * FEAT1, K * S1), lambda i: (0, 0)),
            pl.BlockSpec((K * S1, K), lambda i: (0, 0)),
            pl.BlockSpec((K * S1, S1 * FEAT1), lambda i: (0, 0)),
            pl.BlockSpec((K, FEAT1), lambda i: (0, 0)),
            pl.BlockSpec((S1 * FEAT1, FEAT1), lambda i: (0, 0)),
            pl.BlockSpec((FEAT1, LDIM), lambda i: (0, 0)),
            pl.BlockSpec((LDIM, 1), lambda i: (0, 0)),
            pl.BlockSpec((LDIM, 1), lambda i: (0, 0)),
            pl.BlockSpec((LDIM, 1), lambda i: (0, 0)),
            pl.BlockSpec((1, 1), lambda i: (0, 0)),
        ],
        out_specs=pl.BlockSpec((bb, 1), lambda i: (i, 0)),
        out_shape=jax.ShapeDtypeStruct((B, 1), jnp.float32),
        scratch_shapes=[
            pltpu.VMEM((S1 * bb * S0, NFEAT), jnp.float32),
            pltpu.SemaphoreType.DMA((S1,)),
        ],
    )(nodes, node_neis, node_second_neis, w1m, a1lrep, r_m, sum4, bcast, bc4, slabsum,
      w2m, a2l, a2r, Wl, bl2)
    return out


# speedup vs baseline: 1.3662x; 1.3662x over previous
import functools

import jax
import jax.numpy as jnp
from jax.experimental import pallas as pl
from jax.experimental.pallas import tpu as pltpu

K = 4
NHID = 16
NFEAT = 128
S0 = 25
S1 = 10
B = 2048
LDIM = 32
FEAT1 = K * NHID


def _leaky(x):
    return jnp.where(x >= 0, x, 0.2 * x)


def _gat_kernel(nodes_ref, neis_ref, sneis_hbm, w1m_ref, a1lrep_ref, r_ref,
                sum4_ref, bcast_ref, bc4_ref, slabsum_ref,
                w2m_ref, a2l_ref, a2r_ref, wl_ref, bl_ref, out_ref,
                slab_ref, sems, *, bb, nblocks):
    n1 = bb * S0
    f32 = jnp.float32
    i = pl.program_id(0)

    @pl.when(i < nblocks)
    def _():
        for s in range(S1):
            pltpu.make_async_copy(
                sneis_hbm.at[pl.ds(i * n1, n1), s, :],
                slab_ref.at[i % 2, pl.ds(s * n1, n1), :],
                sems.at[i % 2, s],
            ).start()

    @pl.when(i > 0)
    def _():
        prev = i - 1
        pp = prev % 2
        for s in range(S1):
            pltpu.make_async_copy(
                sneis_hbm.at[pl.ds(prev * n1, n1), s, :],
                slab_ref.at[pp, pl.ds(s * n1, n1), :],
                sems.at[pp, s],
            ).wait()

    w1m = w1m_ref[...]
    pp = jnp.maximum(i - 1, 0) % 2

    hn_s = [
        jnp.dot(slab_ref[pp, pl.ds(s * n1, n1), :], w1m,
                preferred_element_type=f32)
        for s in range(S1)
    ]
    hnw = jnp.concatenate(hn_s, axis=1)

    es = jnp.dot(neis_ref[...], a1lrep_ref[...], preferred_element_type=f32)
    en = jnp.dot(hnw, r_ref[...], preferred_element_type=f32)
    e = _leaky(es + en)
    p = jnp.exp(jnp.minimum(e, 60.0))

    den = jnp.dot(p, sum4_ref[...], preferred_element_type=f32)
    rden = 1.0 / den
    rdenb = jnp.dot(rden, bc4_ref[...], preferred_element_type=f32)

    pb = jnp.dot(p, bcast_ref[...], preferred_element_type=f32)
    weighted = pb * hnw
    acc = jnp.dot(weighted, slabsum_ref[...], preferred_element_type=f32)

    out1 = acc * rdenb
    g1 = jnp.where(out1 > 0, out1, jnp.exp(jnp.minimum(out1, 0.0)) - 1.0)

    w2m = w2m_ref[...]
    hs2 = jnp.dot(nodes_ref[...], w2m, preferred_element_type=f32)
    hn2 = jnp.dot(g1, w2m, preferred_element_type=f32)

    es2 = jnp.dot(hs2, a2l_ref[...], preferred_element_type=f32)
    en2 = jnp.dot(hn2, a2r_ref[...], preferred_element_type=f32)

    nid = jax.lax.broadcasted_iota(jnp.int32, (bb, n1), 1) // S0
    bid = jax.lax.broadcasted_iota(jnp.int32, (bb, n1), 0)
    seg = jnp.where(nid == bid, 1.0, 0.0)
    rmod = jax.lax.broadcasted_iota(jnp.int32, (n1, S0), 0) % S0
    jidx = jax.lax.broadcasted_iota(jnp.int32, (n1, S0), 1)
    t = jnp.where(rmod == jidx, 1.0, 0.0)

    e2m = jnp.dot(seg, en2 * t, preferred_element_type=f32)
    e2 = _leaky(es2 + e2m)
    p2 = jnp.exp(jnp.minimum(e2, 60.0))
    alpha2 = p2 / jnp.sum(p2, axis=1, keepdims=True)

    alpha_tiled = jnp.concatenate([alpha2] * bb, axis=1)
    seg_w = alpha_tiled * seg
    out2 = jnp.dot(seg_w, hn2, preferred_element_type=f32)

    z = jnp.dot(out2, wl_ref[...], preferred_element_type=f32) + bl_ref[...]
    out_ref[...] = 1.0 / (1.0 + jnp.exp(-z))


def kernel(nodes, node_neis, node_second_neis, W1, a1, W2, a2, Wl, bl):
    bb = 32
    nblocks = B // bb
    grid = (nblocks + 1,)
    f32 = jnp.float32

    w1m = jnp.transpose(W1, (1, 0, 2)).reshape(NFEAT, FEAT1)
    a1l = a1[:, :NHID]
    a1r = a1[:, NHID:]
    eye_k = jnp.eye(K, dtype=f32)
    eye_s = jnp.eye(S1, dtype=f32)

    u = jnp.einsum('kfo,ko->fk', W1, a1l)
    erep = jnp.repeat(eye_k, S1, axis=0).T
    a1lrep = u @ erep

    r6 = (a1r[None, :, :, None, None]
          * eye_k[None, :, None, :, None]
          * eye_s[:, None, None, None, :])
    r_m = r6.reshape(S1 * FEAT1, K * S1)

    sum4 = jnp.repeat(eye_k, S1, axis=0)
    y = (eye_k[:, None, None, :, None]
         * eye_s[None, :, :, None, None]
         * jnp.ones((1, 1, 1, 1, NHID), f32))
    bcast = y.reshape(K * S1, S1 * FEAT1)
    bc4 = jnp.repeat(eye_k, NHID, axis=1)
    slabsum = jnp.tile(jnp.eye(FEAT1, dtype=f32), (S1, 1))

    w2m = W2[0]
    a2l = a2[0, :LDIM].reshape(LDIM, 1)
    a2r = a2[0, LDIM:].reshape(LDIM, 1)
    bl2 = bl.reshape(1, 1)

    lag = lambda i: (jnp.maximum(i - 1, 0), 0)

    out = pl.pallas_call(
        functools.partial(_gat_kernel, bb=bb, nblocks=nblocks),
        grid=grid,
        in_specs=[
            pl.BlockSpec((bb, FEAT1), lag),
            pl.BlockSpec((bb * S0, NFEAT), lag),
            pl.BlockSpec(memory_space=pl.ANY),
            pl.BlockSpec((NFEAT, FEAT1), lambda i: (0, 0)),
            pl.BlockSpec((NFEAT, K * S1), lambda i: (0, 0)),
            pl.BlockSpec((S1 * FEAT1, K * S1), lambda i: (0, 0)),
            pl.BlockSpec((K * S1, K), lambda i: (0, 0)),
            pl.BlockSpec((K * S1, S1 * FEAT1), lambda i: (0, 0)),
            pl.BlockSpec((K, FEAT1), lambda i: (0, 0)),
            pl.BlockSpec((S1 * FEAT1, FEAT1), lambda i: (0, 0)),
            pl.BlockSpec((FEAT1, LDIM), lambda i: (0, 0)),
            pl.BlockSpec((LDIM, 1), lambda i: (0, 0)),
            pl.BlockSpec((LDIM, 1), lambda i: (0, 0)),
            pl.BlockSpec((LDIM, 1), lambda i: (0, 0)),
            pl.BlockSpec((1, 1), lambda i: (0, 0)),
        ],
        out_specs=pl.BlockSpec((bb, 1), lag),
        out_shape=jax.ShapeDtypeStruct((B, 1), jnp.float32),
        scratch_shapes=[
            pltpu.VMEM((2, S1 * bb * S0, NFEAT), jnp.float32),
            pltpu.SemaphoreType.DMA((2, S1)),
        ],
    )(nodes, node_neis, node_second_neis, w1m, a1lrep, r_m, sum4, bcast, bc4,
      slabsum, w2m, a2l, a2r, Wl, bl2)
    return out


# speedup vs baseline: 1.4288x; 1.0458x over previous
import functools

import jax
import jax.numpy as jnp
from jax.experimental import pallas as pl
from jax.experimental.pallas import tpu as pltpu

K = 4
NHID = 16
NFEAT = 128
S0 = 25
S1 = 10
B = 2048
LDIM = 32
FEAT1 = K * NHID


def _leaky(x):
    return jnp.where(x >= 0, x, 0.2 * x)


def _gat_kernel(nodes_ref, neis_ref, sneis_hbm, w1m_ref, a1lrep_ref, r_ref,
                sum4_ref, bcast_ref, bc4_ref, slabsum_ref,
                w2m_ref, a2l_ref, a2r_ref, wl_ref, bl_ref, out_ref,
                slab_ref, sems, *, bb, nblocks):
    n1 = bb * S0
    f32 = jnp.float32
    i = pl.program_id(0)

    @pl.when(i < nblocks)
    def _():
        for s in range(S1):
            pltpu.make_async_copy(
                sneis_hbm.at[pl.ds(i * n1, n1), s, :],
                slab_ref.at[i % 2, pl.ds(s * n1, n1), :],
                sems.at[i % 2, s],
            ).start()

    @pl.when(i > 0)
    def _():
        prev = i - 1
        pp = prev % 2
        for s in range(S1):
            pltpu.make_async_copy(
                sneis_hbm.at[pl.ds(prev * n1, n1), s, :],
                slab_ref.at[pp, pl.ds(s * n1, n1), :],
                sems.at[pp, s],
            ).wait()

    w1m = w1m_ref[...]
    pp = jnp.maximum(i - 1, 0) % 2

    hn_s = [
        jnp.dot(slab_ref[pp, pl.ds(s * n1, n1), :], w1m,
                preferred_element_type=f32)
        for s in range(S1)
    ]
    hnw = jnp.concatenate(hn_s, axis=1)

    es = jnp.dot(neis_ref[...], a1lrep_ref[...], preferred_element_type=f32)
    en = jnp.dot(hnw, r_ref[...], preferred_element_type=f32)
    e = _leaky(es + en)
    p = jnp.exp(jnp.minimum(e, 60.0))

    den = jnp.dot(p, sum4_ref[...], preferred_element_type=f32)
    rden = 1.0 / den
    rdenb = jnp.dot(rden, bc4_ref[...], preferred_element_type=f32)

    pb = jnp.dot(p, bcast_ref[...], preferred_element_type=f32)
    weighted = pb * hnw
    acc = jnp.dot(weighted, slabsum_ref[...], preferred_element_type=f32)

    out1 = acc * rdenb
    g1 = jnp.where(out1 > 0, out1, jnp.exp(jnp.minimum(out1, 0.0)) - 1.0)

    w2m = w2m_ref[...]
    hs2 = jnp.dot(nodes_ref[...], w2m, preferred_element_type=f32)
    hn2 = jnp.dot(g1, w2m, preferred_element_type=f32)

    es2 = jnp.dot(hs2, a2l_ref[...], preferred_element_type=f32)
    en2 = jnp.dot(hn2, a2r_ref[...], preferred_element_type=f32)

    nid = jax.lax.broadcasted_iota(jnp.int32, (bb, n1), 1) // S0
    bid = jax.lax.broadcasted_iota(jnp.int32, (bb, n1), 0)
    seg = jnp.where(nid == bid, 1.0, 0.0)
    rmod = jax.lax.broadcasted_iota(jnp.int32, (n1, S0), 0) % S0
    jidx = jax.lax.broadcasted_iota(jnp.int32, (n1, S0), 1)
    t = jnp.where(rmod == jidx, 1.0, 0.0)

    e2m = jnp.dot(seg, en2 * t, preferred_element_type=f32)
    e2 = _leaky(es2 + e2m)
    p2 = jnp.exp(jnp.minimum(e2, 60.0))
    alpha2 = p2 / jnp.sum(p2, axis=1, keepdims=True)

    alpha_tiled = jnp.concatenate([alpha2] * bb, axis=1)
    seg_w = alpha_tiled * seg
    out2 = jnp.dot(seg_w, hn2, preferred_element_type=f32)

    z = jnp.dot(out2, wl_ref[...], preferred_element_type=f32) + bl_ref[...]
    out_ref[...] = 1.0 / (1.0 + jnp.exp(-z))


def kernel(nodes, node_neis, node_second_neis, W1, a1, W2, a2, Wl, bl):
    bb = 64
    nblocks = B // bb
    grid = (nblocks + 1,)
    f32 = jnp.float32

    w1m = jnp.transpose(W1, (1, 0, 2)).reshape(NFEAT, FEAT1)
    a1l = a1[:, :NHID]
    a1r = a1[:, NHID:]
    eye_k = jnp.eye(K, dtype=f32)
    eye_s = jnp.eye(S1, dtype=f32)

    u = jnp.einsum('kfo,ko->fk', W1, a1l)
    erep = jnp.repeat(eye_k, S1, axis=0).T
    a1lrep = u @ erep

    r6 = (a1r[None, :, :, None, None]
          * eye_k[None, :, None, :, None]
          * eye_s[:, None, None, None, :])
    r_m = r6.reshape(S1 * FEAT1, K * S1)

    sum4 = jnp.repeat(eye_k, S1, axis=0)
    y = (eye_k[:, None, None, :, None]
         * eye_s[None, :, :, None, None]
         * jnp.ones((1, 1, 1, 1, NHID), f32))
    bcast = y.reshape(K * S1, S1 * FEAT1)
    bc4 = jnp.repeat(eye_k, NHID, axis=1)
    slabsum = jnp.tile(jnp.eye(FEAT1, dtype=f32), (S1, 1))

    w2m = W2[0]
    a2l = a2[0, :LDIM].reshape(LDIM, 1)
    a2r = a2[0, LDIM:].reshape(LDIM, 1)
    bl2 = bl.reshape(1, 1)

    lag = lambda i: (jnp.maximum(i - 1, 0), 0)

    out = pl.pallas_call(
        functools.partial(_gat_kernel, bb=bb, nblocks=nblocks),
        grid=grid,
        in_specs=[
            pl.BlockSpec((bb, FEAT1), lag),
            pl.BlockSpec((bb * S0, NFEAT), lag),
            pl.BlockSpec(memory_space=pl.ANY),
            pl.BlockSpec((NFEAT, FEAT1), lambda i: (0, 0)),
            pl.BlockSpec((NFEAT, K * S1), lambda i: (0, 0)),
            pl.BlockSpec((S1 * FEAT1, K * S1), lambda i: (0, 0)),
            pl.BlockSpec((K * S1, K), lambda i: (0, 0)),
            pl.BlockSpec((K * S1, S1 * FEAT1), lambda i: (0, 0)),
            pl.BlockSpec((K, FEAT1), lambda i: (0, 0)),
            pl.BlockSpec((S1 * FEAT1, FEAT1), lambda i: (0, 0)),
            pl.BlockSpec((FEAT1, LDIM), lambda i: (0, 0)),
            pl.BlockSpec((LDIM, 1), lambda i: (0, 0)),
            pl.BlockSpec((LDIM, 1), lambda i: (0, 0)),
            pl.BlockSpec((LDIM, 1), lambda i: (0, 0)),
            pl.BlockSpec((1, 1), lambda i: (0, 0)),
        ],
        out_specs=pl.BlockSpec((bb, 1), lag),
        out_shape=jax.ShapeDtypeStruct((B, 1), jnp.float32),
        scratch_shapes=[
            pltpu.VMEM((2, S1 * bb * S0, NFEAT), jnp.float32),
            pltpu.SemaphoreType.DMA((2, S1)),
        ],
    )(nodes, node_neis, node_second_neis, w1m, a1lrep, r_m, sum4, bcast, bc4,
      slabsum, w2m, a2l, a2r, Wl, bl2)
    return out
